# baseline (speedup 1.0000x reference)
import functools

import jax
import jax.numpy as jnp
from jax import lax
from jax.experimental import pallas as pl
from jax.experimental.pallas import tpu as pltpu
from jax.experimental.pallas import tpu_sc as plsc

N_R, D = 100000, 128
NC, NS = 2, 16
NW = NC * NS
CH = 128
NCH0 = 50
NCH1 = 0
NCHS = NS * NCH0
NCHT = NS * (NCH0 + NCH1)
N_PAD = NCHT * CH
NB = 7


@functools.lru_cache(maxsize=1)
def _make_gather():
    mesh = plsc.VectorSubcoreMesh(
        core_axis_name="c", subcore_axis_name="s", num_cores=NC, num_subcores=NS)

    @functools.partial(
        pl.kernel,
        out_type=(jax.ShapeDtypeStruct((N_PAD, D), jnp.float32),
                  jax.ShapeDtypeStruct((N_PAD, D), jnp.float32)),
        mesh=mesh,
        scratch_types=(
            [pltpu.VMEM((NCH0, CH), jnp.int32)] * 2
            + [pltpu.VMEM((CH, D), jnp.float32)] * NB
            + [pltpu.SemaphoreType.DMA] * (2 * NB)
        ),
    )
    def gather_k(tab_u, tab_i, idx_u, idx_i, out_u, out_i, *rest):
        iv_u, iv_i = rest[0], rest[1]
        bufs = rest[2:2 + NB]
        gs = rest[2 + NB:2 + 2 * NB]
        ss = rest[2 + 2 * NB:2 + 3 * NB]
        c = lax.axis_index("c")
        s = lax.axis_index("s")
        wid = c * NS + s
        pltpu.sync_copy(idx_u.at[wid], iv_u)
        pltpu.sync_copy(idx_i.at[wid], iv_i)
        tabs, ivs, outs = (tab_u, tab_i), (iv_u, iv_i), (out_u, out_i)

        def run(nch, base_chunk):
            base = base_chunk * CH
            NT = 2 * nch

            def win(w):
                t, j = w & 1, w >> 1
                return tabs[t], ivs[t], outs[t], pl.multiple_of(base + j * CH, CH)

            gcp, scp = {}, {}
            waited = set()
            for w in range(min(NB - 1, NT)):
                tab, iv, out, row0 = win(w)
                gcp[w] = pltpu.async_copy(
                    tab.at[iv.at[w >> 1]], bufs[w % NB], gs[w % NB])
            for w in range(NT):
                b = w % NB
                tab, iv, out, row0 = win(w)
                gcp[w].wait()
                scp[w] = pltpu.async_copy(bufs[b], out.at[pl.ds(row0, CH)], ss[b])
                nw = w + NB - 1
                if nw < NT:
                    if w >= 1:
                        scp[w - 1].wait()
                        waited.add(w - 1)
                    ntab, niv, nout, _ = win(nw)
                    gcp[nw] = pltpu.async_copy(
                        ntab.at[niv.at[nw >> 1]], bufs[nw % NB], gs[nw % NB])
            for w in range(NT):
                if w not in waited:
                    scp[w].wait()

        @pl.when(c == 0)
        def _():
            run(NCH0, s * NCH0)

        if NCH1:
            @pl.when(c == 1)
            def _():
                run(NCH1, NCHS + s * NCH1)

    return gather_k


BR = 1000


def _mm_body(r_ref, u_ref, i_ref, w_ref, o_ref):
    acc = jnp.dot(r_ref[...], w_ref[0:D, :], preferred_element_type=jnp.float32)
    acc += jnp.dot(u_ref[...], w_ref[D:2 * D, :], preferred_element_type=jnp.float32)
    acc += jnp.dot(i_ref[...], w_ref[2 * D:3 * D, :], preferred_element_type=jnp.float32)
    o_ref[...] = jnp.maximum(acc, 0.0)


def _matmul_relu(review, ru, ri, w):
    return pl.pallas_call(
        _mm_body,
        grid=(N_R // BR,),
        in_specs=[
            pl.BlockSpec((BR, D), lambda i: (i, 0)),
            pl.BlockSpec((BR, D), lambda i: (i, 0)),
            pl.BlockSpec((BR, D), lambda i: (i, 0)),
            pl.BlockSpec((3 * D, D), lambda i: (0, 0)),
        ],
        out_specs=pl.BlockSpec((BR, D), lambda i: (i, 0)),
        out_shape=jax.ShapeDtypeStruct((N_R, D), jnp.float32),
    )(review, ru, ri, w)


def kernel(review_vecs, user_vecs, item_vecs, review_item_adj, review_user_adj, con_agg_weights):
    perm_i = jax.random.permutation(jax.random.key(1), D)
    perm_u = jax.random.permutation(jax.random.key(2), D)
    wr = con_agg_weights[:D]
    wu = con_agg_weights[D:2 * D][jnp.argsort(perm_u)]
    wi = con_agg_weights[2 * D:][jnp.argsort(perm_i)]
    w = jnp.concatenate([wr, wu, wi], axis=0)

    def pad_idx(a):
        f = jnp.zeros((N_PAD,), jnp.int32).at[:N_R].set(a).reshape(NCHT, CH)
        big = jnp.zeros((NW, NCH0, CH), jnp.int32)
        big = big.at[:NS].set(f[:NCHS].reshape(NS, NCH0, CH))
        if NCH1:
            big = big.at[NS:, :NCH1].set(f[NCHS:].reshape(NS, NCH1, CH))
        return big

    gather = _make_gather()
    ru, ri = gather(user_vecs, item_vecs,
                    pad_idx(review_user_adj), pad_idx(review_item_adj))
    return _matmul_relu(review_vecs, ru, ri, w)


# speedup vs baseline: 1.0351x; 1.0351x over previous
import functools

import jax
import jax.numpy as jnp
from jax import lax
from jax.experimental import pallas as pl
from jax.experimental.pallas import tpu as pltpu
from jax.experimental.pallas import tpu_sc as plsc

N_R, D = 100000, 128
NC, NS = 2, 16
NW = NC * NS
CH = 128
NCH0 = 36
NCH1 = 14
NCHS = NS * NCH0
NCHT = NS * (NCH0 + NCH1)
N_PAD = NCHT * CH
NB = 7


@functools.lru_cache(maxsize=1)
def _make_gather():
    mesh = plsc.VectorSubcoreMesh(
        core_axis_name="c", subcore_axis_name="s", num_cores=NC, num_subcores=NS)

    @functools.partial(
        pl.kernel,
        out_type=(jax.ShapeDtypeStruct((N_PAD, D), jnp.float32),
                  jax.ShapeDtypeStruct((N_PAD, D), jnp.float32)),
        mesh=mesh,
        scratch_types=(
            [pltpu.VMEM((NCH0, CH), jnp.int32)] * 2
            + [pltpu.VMEM((CH, D), jnp.float32)] * NB
            + [pltpu.SemaphoreType.DMA] * (2 * NB)
        ),
    )
    def gather_k(tab_u, tab_i, idx_u, idx_i, out_u, out_i, *rest):
        iv_u, iv_i = rest[0], rest[1]
        bufs = rest[2:2 + NB]
        gs = rest[2 + NB:2 + 2 * NB]
        ss = rest[2 + 2 * NB:2 + 3 * NB]
        c = lax.axis_index("c")
        s = lax.axis_index("s")
        wid = c * NS + s
        pltpu.sync_copy(idx_u.at[wid], iv_u)
        pltpu.sync_copy(idx_i.at[wid], iv_i)
        tabs, ivs, outs = (tab_u, tab_i), (iv_u, iv_i), (out_u, out_i)

        def run(nch, base_chunk):
            base = base_chunk * CH
            NT = 2 * nch

            def win(w):
                t, j = w & 1, w >> 1
                return tabs[t], ivs[t], outs[t], pl.multiple_of(base + j * CH, CH)

            gcp, scp = {}, {}
            waited = set()
            for w in range(min(NB - 1, NT)):
                tab, iv, out, row0 = win(w)
                gcp[w] = pltpu.async_copy(
                    tab.at[iv.at[w >> 1]], bufs[w % NB], gs[w % NB])
            for w in range(NT):
                b = w % NB
                tab, iv, out, row0 = win(w)
                gcp[w].wait()
                scp[w] = pltpu.async_copy(bufs[b], out.at[pl.ds(row0, CH)], ss[b])
                nw = w + NB - 1
                if nw < NT:
                    if w >= 1:
                        scp[w - 1].wait()
                        waited.add(w - 1)
                    ntab, niv, nout, _ = win(nw)
                    gcp[nw] = pltpu.async_copy(
                        ntab.at[niv.at[nw >> 1]], bufs[nw % NB], gs[nw % NB])
            for w in range(NT):
                if w not in waited:
                    scp[w].wait()

        def run_batched(nch, base_chunk):
            base = base_chunk * CH
            NT = 2 * nch

            def win(w):
                t, j = w & 1, w >> 1
                return tabs[t], ivs[t], outs[t], pl.multiple_of(base + j * CH, CH)

            for w0 in range(0, NT, NB):
                batch = range(w0, min(w0 + NB, NT))
                cps = []
                for w in batch:
                    tab, iv, out, row0 = win(w)
                    cps.append(pltpu.async_copy(
                        tab.at[iv.at[w >> 1]], bufs[w % NB], gs[w % NB]))
                for cp in cps:
                    cp.wait()
                cps = []
                for w in batch:
                    tab, iv, out, row0 = win(w)
                    cps.append(pltpu.async_copy(
                        bufs[w % NB], out.at[pl.ds(row0, CH)], ss[w % NB]))
                for cp in cps:
                    cp.wait()

        @pl.when(c == 0)
        def _():
            run(NCH0, s * NCH0)

        if NCH1:
            @pl.when(c == 1)
            def _():
                run_batched(NCH1, NCHS + s * NCH1)

    return gather_k


BR = 1000


def _mm_body(r_ref, u_ref, i_ref, w_ref, o_ref):
    acc = jnp.dot(r_ref[...], w_ref[0:D, :], preferred_element_type=jnp.float32)
    acc += jnp.dot(u_ref[...], w_ref[D:2 * D, :], preferred_element_type=jnp.float32)
    acc += jnp.dot(i_ref[...], w_ref[2 * D:3 * D, :], preferred_element_type=jnp.float32)
    o_ref[...] = jnp.maximum(acc, 0.0)


def _matmul_relu(review, ru, ri, w):
    return pl.pallas_call(
        _mm_body,
        grid=(N_R // BR,),
        in_specs=[
            pl.BlockSpec((BR, D), lambda i: (i, 0)),
            pl.BlockSpec((BR, D), lambda i: (i, 0)),
            pl.BlockSpec((BR, D), lambda i: (i, 0)),
            pl.BlockSpec((3 * D, D), lambda i: (0, 0)),
        ],
        out_specs=pl.BlockSpec((BR, D), lambda i: (i, 0)),
        out_shape=jax.ShapeDtypeStruct((N_R, D), jnp.float32),
    )(review, ru, ri, w)


def kernel(review_vecs, user_vecs, item_vecs, review_item_adj, review_user_adj, con_agg_weights):
    perm_i = jax.random.permutation(jax.random.key(1), D)
    perm_u = jax.random.permutation(jax.random.key(2), D)
    wr = con_agg_weights[:D]
    wu = con_agg_weights[D:2 * D][jnp.argsort(perm_u)]
    wi = con_agg_weights[2 * D:][jnp.argsort(perm_i)]
    w = jnp.concatenate([wr, wu, wi], axis=0)

    def pad_idx(a):
        f = jnp.zeros((N_PAD,), jnp.int32).at[:N_R].set(a).reshape(NCHT, CH)
        big = jnp.zeros((NW, NCH0, CH), jnp.int32)
        big = big.at[:NS].set(f[:NCHS].reshape(NS, NCH0, CH))
        if NCH1:
            big = big.at[NS:, :NCH1].set(f[NCHS:].reshape(NS, NCH1, CH))
        return big

    gather = _make_gather()
    ru, ri = gather(user_vecs, item_vecs,
                    pad_idx(review_user_adj), pad_idx(review_item_adj))
    return _matmul_relu(review_vecs, ru, ri, w)


# speedup vs baseline: 1.1283x; 1.0900x over previous
import functools

import jax
import jax.numpy as jnp
from jax import lax
from jax.experimental import pallas as pl
from jax.experimental.pallas import tpu as pltpu
from jax.experimental.pallas import tpu_sc as plsc

N_R, D = 100000, 128
NC, NS = 2, 16
NW = NC * NS
CH = 128
NCH0 = 36
NCH1 = 14
NCHS = NS * NCH0
NCHT = NS * (NCH0 + NCH1)
N_PAD = NCHT * CH
NB = 7


@functools.lru_cache(maxsize=2)
def _make_gather(nch0, nch1):
    nchs = NS * nch0
    n_pad = NS * (nch0 + nch1) * CH
    mesh = plsc.VectorSubcoreMesh(
        core_axis_name="c", subcore_axis_name="s", num_cores=NC, num_subcores=NS)

    @functools.partial(
        pl.kernel,
        out_type=(jax.ShapeDtypeStruct((n_pad, D), jnp.float32),
                  jax.ShapeDtypeStruct((n_pad, D), jnp.float32)),
        mesh=mesh,
        scratch_types=(
            [pltpu.VMEM((nch0, CH), jnp.int32)] * 2
            + [pltpu.VMEM((CH, D), jnp.float32)] * NB
            + [pltpu.SemaphoreType.DMA] * (2 * NB)
        ),
    )
    def gather_k(tab_u, tab_i, idx_u, idx_i, out_u, out_i, *rest):
        iv_u, iv_i = rest[0], rest[1]
        bufs = rest[2:2 + NB]
        gs = rest[2 + NB:2 + 2 * NB]
        ss = rest[2 + 2 * NB:2 + 3 * NB]
        c = lax.axis_index("c")
        s = lax.axis_index("s")
        wid = c * NS + s
        pltpu.sync_copy(idx_u.at[wid], iv_u)
        pltpu.sync_copy(idx_i.at[wid], iv_i)
        tabs, ivs, outs = (tab_u, tab_i), (iv_u, iv_i), (out_u, out_i)

        def run(nch, base_chunk):
            base = base_chunk * CH
            NT = 2 * nch

            def win(w):
                t, j = w & 1, w >> 1
                return tabs[t], ivs[t], outs[t], pl.multiple_of(base + j * CH, CH)

            gcp, scp = {}, {}
            waited = set()
            for w in range(min(NB - 1, NT)):
                tab, iv, out, row0 = win(w)
                gcp[w] = pltpu.async_copy(
                    tab.at[iv.at[w >> 1]], bufs[w % NB], gs[w % NB])
            for w in range(NT):
                b = w % NB
                tab, iv, out, row0 = win(w)
                gcp[w].wait()
                scp[w] = pltpu.async_copy(bufs[b], out.at[pl.ds(row0, CH)], ss[b])
                nw = w + NB - 1
                if nw < NT:
                    if w >= 1:
                        scp[w - 1].wait()
                        waited.add(w - 1)
                    ntab, niv, nout, _ = win(nw)
                    gcp[nw] = pltpu.async_copy(
                        ntab.at[niv.at[nw >> 1]], bufs[nw % NB], gs[nw % NB])
            for w in range(NT):
                if w not in waited:
                    scp[w].wait()

        @pl.when(c == 0)
        def _():
            run(nch0, s * nch0)

        @pl.when(c == 1)
        def _():
            run(nch1, nchs + s * nch1)

    return gather_k


BR = 800
H_ROWS = N_PAD // 2
H_BLOCKS = H_ROWS // BR
H2_BLOCKS = (N_R - H_ROWS) // BR


def _mm_body(r_ref, u_ref, i_ref, w_ref, o_ref):
    acc = jnp.dot(r_ref[...], w_ref[0:D, :], preferred_element_type=jnp.float32)
    acc += jnp.dot(u_ref[...], w_ref[D:2 * D, :], preferred_element_type=jnp.float32)
    acc += jnp.dot(i_ref[...], w_ref[2 * D:3 * D, :], preferred_element_type=jnp.float32)
    o_ref[...] = jnp.maximum(acc, 0.0)


def _mm_body2(r_ref, u_ref, i_ref, w_ref, prev_ref, o_ref):
    _mm_body(r_ref, u_ref, i_ref, w_ref, o_ref)


def _matmul_relu_h1(review, ru, ri, w):
    return pl.pallas_call(
        _mm_body,
        grid=(H_BLOCKS,),
        in_specs=[
            pl.BlockSpec((BR, D), lambda i: (i, 0)),
            pl.BlockSpec((BR, D), lambda i: (i, 0)),
            pl.BlockSpec((BR, D), lambda i: (i, 0)),
            pl.BlockSpec((3 * D, D), lambda i: (0, 0)),
        ],
        out_specs=pl.BlockSpec((BR, D), lambda i: (i, 0)),
        out_shape=jax.ShapeDtypeStruct((N_R, D), jnp.float32),
    )(review, ru, ri, w)


def _matmul_relu_h2(review, ru, ri, w, prev):
    return pl.pallas_call(
        _mm_body2,
        grid=(H2_BLOCKS,),
        in_specs=[
            pl.BlockSpec((BR, D), lambda i: (i + H_BLOCKS, 0)),
            pl.BlockSpec((BR, D), lambda i: (i, 0)),
            pl.BlockSpec((BR, D), lambda i: (i, 0)),
            pl.BlockSpec((3 * D, D), lambda i: (0, 0)),
            pl.BlockSpec(memory_space=pl.ANY),
        ],
        out_specs=pl.BlockSpec((BR, D), lambda i: (i + H_BLOCKS, 0)),
        out_shape=jax.ShapeDtypeStruct((N_R, D), jnp.float32),
        input_output_aliases={4: 0},
    )(review, ru, ri, w, prev)


def kernel(review_vecs, user_vecs, item_vecs, review_item_adj, review_user_adj, con_agg_weights):
    perm_i = jax.random.permutation(jax.random.key(1), D)
    perm_u = jax.random.permutation(jax.random.key(2), D)
    wr = con_agg_weights[:D]
    wu = con_agg_weights[D:2 * D][jnp.argsort(perm_u)]
    wi = con_agg_weights[2 * D:][jnp.argsort(perm_i)]
    w = jnp.concatenate([wr, wu, wi], axis=0)

    nch0h, nch1h = NCH0 // 2, NCH1 // 2
    ncht_h = NS * (nch0h + nch1h)
    nchs_h = NS * nch0h

    def pad_idx(a):
        f = jnp.zeros((N_PAD,), jnp.int32).at[:N_R].set(a).reshape(NCHT, CH)

        def half(fh):
            big = jnp.zeros((NW, nch0h, CH), jnp.int32)
            big = big.at[:NS].set(fh[:nchs_h].reshape(NS, nch0h, CH))
            big = big.at[NS:, :nch1h].set(fh[nchs_h:].reshape(NS, nch1h, CH))
            return big

        return half(f[:ncht_h]), half(f[ncht_h:])

    gather = _make_gather(nch0h, nch1h)
    iu1, iu2 = pad_idx(review_user_adj)
    ii1, ii2 = pad_idx(review_item_adj)
    ru1, ri1 = gather(user_vecs, item_vecs, iu1, ii1)
    ru2, ri2 = gather(user_vecs, item_vecs, iu2, ii2)
    out = _matmul_relu_h1(review_vecs, ru1, ri1, w)
    return _matmul_relu_h2(review_vecs, ru2, ri2, w, out)
